# Initial kernel scaffold; baseline (speedup 1.0000x reference)
import jax
import jax.numpy as jnp
from jax.experimental import pallas as pl


def kernel(x, edge_index, edge_attr, batch, Wl1, bl1, Wr1, br1, We1, att1, bias1, Wl2, bl2, Wr2, br2, We2, att2, bias2, W3, b3):
    raise NotImplementedError("write your pallas kernel here")


# speedup vs baseline: 4.5761x; 4.5761x over previous
import functools

import jax
import jax.numpy as jnp
from jax.experimental import pallas as pl


def _proj_body(x_ref, w_ref, b_ref, o_ref):
    o_ref[...] = (
        jnp.dot(x_ref[...], w_ref[...], preferred_element_type=jnp.float32)
        + b_ref[...]
    )


def _proj(x, w, b, bn):
    n, k = x.shape
    m = w.shape[1]
    return pl.pallas_call(
        _proj_body,
        grid=(n // bn,),
        in_specs=[
            pl.BlockSpec((bn, k), lambda i: (i, 0)),
            pl.BlockSpec((k, m), lambda i: (0, 0)),
            pl.BlockSpec((1, m), lambda i: (0, 0)),
        ],
        out_specs=pl.BlockSpec((bn, m), lambda i: (i, 0)),
        out_shape=jax.ShapeDtypeStruct((n, m), jnp.float32),
    )(x, w, b.reshape(1, m))


def _alpha_body(xls_ref, xrd_ref, ea_ref, we_ref, attm_ref, o_ref):
    z = xls_ref[...] + xrd_ref[...] + ea_ref[...] * we_ref[...]
    z = jnp.where(z >= 0.0, z, 0.2 * z)
    o_ref[...] = jnp.dot(z, attm_ref[...], preferred_element_type=jnp.float32)


def _edge_alpha(xls, xrd, ea, we, attm, be):
    e, m = xls.shape
    h = attm.shape[1]
    return pl.pallas_call(
        _alpha_body,
        grid=(e // be,),
        in_specs=[
            pl.BlockSpec((be, m), lambda i: (i, 0)),
            pl.BlockSpec((be, m), lambda i: (i, 0)),
            pl.BlockSpec((be, 1), lambda i: (i, 0)),
            pl.BlockSpec((1, m), lambda i: (0, 0)),
            pl.BlockSpec((m, h), lambda i: (0, 0)),
        ],
        out_specs=pl.BlockSpec((be, h), lambda i: (i, 0)),
        out_shape=jax.ShapeDtypeStruct((e, h), jnp.float32),
    )(xls, xrd, ea, we, attm)


def _exp_body(a_ref, m_ref, o_ref):
    o_ref[...] = jnp.exp(a_ref[...] - m_ref[...])


def _edge_exp(alpha, mdst, be):
    e, h = alpha.shape
    return pl.pallas_call(
        _exp_body,
        grid=(e // be,),
        in_specs=[
            pl.BlockSpec((be, h), lambda i: (i, 0)),
            pl.BlockSpec((be, h), lambda i: (i, 0)),
        ],
        out_specs=pl.BlockSpec((be, h), lambda i: (i, 0)),
        out_shape=jax.ShapeDtypeStruct((e, h), jnp.float32),
    )(alpha, mdst)


def _msg_body(xls_ref, e_ref, s_ref, bc_ref, o_ref):
    w = e_ref[...] / (s_ref[...] + 1e-16)
    o_ref[...] = xls_ref[...] * jnp.dot(
        w, bc_ref[...], preferred_element_type=jnp.float32
    )


def _edge_msg(xls, ealpha, sdst, bcast, be):
    e, m = xls.shape
    h = ealpha.shape[1]
    return pl.pallas_call(
        _msg_body,
        grid=(e // be,),
        in_specs=[
            pl.BlockSpec((be, m), lambda i: (i, 0)),
            pl.BlockSpec((be, h), lambda i: (i, 0)),
            pl.BlockSpec((be, h), lambda i: (i, 0)),
            pl.BlockSpec((h, m), lambda i: (0, 0)),
        ],
        out_specs=pl.BlockSpec((be, m), lambda i: (i, 0)),
        out_shape=jax.ShapeDtypeStruct((e, m), jnp.float32),
    )(xls, ealpha, sdst, bcast)


def _bias_elu_body(t_ref, b_ref, o_ref):
    v = t_ref[...] + b_ref[...]
    o_ref[...] = jnp.where(v > 0.0, v, jnp.exp(jnp.minimum(v, 0.0)) - 1.0)


def _bias_elu(t, b, bn):
    n, m = t.shape
    return pl.pallas_call(
        _bias_elu_body,
        grid=(n // bn,),
        in_specs=[
            pl.BlockSpec((bn, m), lambda i: (i, 0)),
            pl.BlockSpec((1, m), lambda i: (0, 0)),
        ],
        out_specs=pl.BlockSpec((bn, m), lambda i: (i, 0)),
        out_shape=jax.ShapeDtypeStruct((n, m), jnp.float32),
    )(t, b.reshape(1, m))


def _pool_body(nsteps, bn, g, h_ref, b_ref, w3_ref, b3_ref, sum_ref, cnt_ref, o_ref):
    i = pl.program_id(0)

    @pl.when(i == 0)
    def _():
        sum_ref[...] = jnp.zeros_like(sum_ref)
        cnt_ref[...] = jnp.zeros_like(cnt_ref)

    oh = (
        b_ref[...] == jax.lax.broadcasted_iota(jnp.int32, (bn, g), 1)
    ).astype(jnp.float32)
    dn = (((0,), (0,)), ((), ()))
    sum_ref[...] += jax.lax.dot_general(
        oh, h_ref[...], dn, preferred_element_type=jnp.float32
    )
    cnt_ref[...] += jax.lax.dot_general(
        oh, jnp.ones_like(h_ref[...]), dn, preferred_element_type=jnp.float32
    )

    @pl.when(i == nsteps - 1)
    def _():
        pooled = sum_ref[...] / jnp.maximum(cnt_ref[...], 1.0)
        o_ref[...] = (
            jnp.dot(pooled, w3_ref[...], preferred_element_type=jnp.float32)
            + b3_ref[...]
        )


def _pool_head(h, batch, w3, b3, g):
    n, m = h.shape
    n_pad = 51200
    bn = 6400
    nsteps = n_pad // bn
    h_p = jnp.pad(h, ((0, n_pad - n), (0, 0)))
    b_p = jnp.pad(batch.astype(jnp.int32), (0, n_pad - n), constant_values=g)
    outs = pl.pallas_call(
        functools.partial(_pool_body, nsteps, bn, g),
        grid=(nsteps,),
        in_specs=[
            pl.BlockSpec((bn, m), lambda i: (i, 0)),
            pl.BlockSpec((bn, 1), lambda i: (i, 0)),
            pl.BlockSpec((m, 1), lambda i: (0, 0)),
            pl.BlockSpec((1, 1), lambda i: (0, 0)),
        ],
        out_specs=[
            pl.BlockSpec((g, m), lambda i: (0, 0)),
            pl.BlockSpec((g, m), lambda i: (0, 0)),
            pl.BlockSpec((g, 1), lambda i: (0, 0)),
        ],
        out_shape=[
            jax.ShapeDtypeStruct((g, m), jnp.float32),
            jax.ShapeDtypeStruct((g, m), jnp.float32),
            jax.ShapeDtypeStruct((g, 1), jnp.float32),
        ],
    )(h_p, b_p.reshape(n_pad, 1), w3, b3.reshape(1, 1))
    return outs[2]


def _block_sizes(e_full, n):
    be = 6800
    while e_full % be != 0:
        be //= 2
        if be < 8:
            be = 1
            break
    bn = 2000
    while n % bn != 0:
        bn //= 2
        if bn < 8:
            bn = 1
            break
    return be, bn


def _gat_layer(x_in, srcf, dstf, ea, wl, bl, wr, br, we, att, bias, heads, ch, n, be, bn):
    m = heads * ch
    xl = _proj(x_in, wl, bl, bn)
    xr = _proj(x_in, wr, br, bn)
    xls = xl[srcf]
    xrd = xr[dstf]
    head_of = jnp.arange(m, dtype=jnp.int32) // ch
    eye = (head_of[:, None] == jnp.arange(heads, dtype=jnp.int32)[None, :]).astype(
        jnp.float32
    )
    attm = att.reshape(-1)[:, None] * eye
    alpha = _edge_alpha(xls, xrd, ea, we, attm, be)
    mseg = jax.ops.segment_max(alpha, dstf, num_segments=n)
    mseg = jnp.where(jnp.isfinite(mseg), mseg, 0.0)
    ealpha = _edge_exp(alpha, mseg[dstf], be)
    sseg = jax.ops.segment_sum(ealpha, dstf, num_segments=n)
    msg = _edge_msg(xls, ealpha, sseg[dstf], eye.T, be)
    out = jax.ops.segment_sum(msg, dstf, num_segments=n)
    return _bias_elu(out, bias, bn)


def kernel(x, edge_index, edge_attr, batch, Wl1, bl1, Wr1, br1, We1, att1, bias1,
           Wl2, bl2, Wr2, br2, We2, att2, bias2, W3, b3):
    n = x.shape[0]
    src = edge_index[0]
    dst = edge_index[1]
    e = src.shape[0]
    g = 64

    deg = jax.ops.segment_sum(jnp.ones((e,), jnp.float32), dst, num_segments=n)
    loop_attr = (
        jax.ops.segment_sum(edge_attr, dst, num_segments=n)
        / jnp.maximum(deg, 1.0)[:, None]
    )
    ar = jnp.arange(n, dtype=src.dtype)
    srcf = jnp.concatenate([src, ar])
    dstf = jnp.concatenate([dst, ar])
    ea = jnp.concatenate([edge_attr, loop_attr], axis=0)

    be, bn = _block_sizes(e + n, n)

    h1 = _gat_layer(
        x, srcf, dstf, ea, Wl1, bl1, Wr1, br1, We1, att1, bias1, 8, 16, n, be, bn
    )
    h2 = _gat_layer(
        h1, srcf, dstf, ea, Wl2, bl2, Wr2, br2, We2, att2, bias2, 1, 8, n, be, bn
    )
    return _pool_head(h2, batch, W3, b3, g)
